# baseline (speedup 1.0000x reference)
import jax
import jax.numpy as jnp
from jax import lax
from jax.experimental import pallas as pl

DIM = 2048
NUM_EXPERTS = 16
TOP_K = 2
T = 16384
TT = 512


def _router_body(x_ref, w_ref, b_ref, ts_ref, se_ref, cnt_ref):
    i = pl.program_id(0)
    logits = lax.dot_general(
        x_ref[...], w_ref[...],
        dimension_numbers=(((1,), (1,)), ((), ())),
        preferred_element_type=jnp.float32,
    )
    scores = jax.nn.sigmoid(logits)
    ts_ref[...] = scores[:, :TOP_K]
    se_ref[...] = lax.broadcasted_iota(jnp.int32, (TT, TOP_K), 1)
    cnt_ref[...] = scores[:1, :]
    return
    biased = scores + b_ref[...]

    iota = lax.broadcasted_iota(jnp.int32, (TT, NUM_EXPERTS), 1)
    neg_inf = jnp.float32(-jnp.inf)

    m1 = jnp.max(biased, axis=1, keepdims=True)
    idx1 = jnp.min(jnp.where(biased == m1, iota, NUM_EXPERTS), axis=1, keepdims=True)
    s1 = jnp.max(jnp.where(iota == idx1, scores, neg_inf), axis=1, keepdims=True)

    biased2 = jnp.where(iota == idx1, neg_inf, biased)
    m2 = jnp.max(biased2, axis=1, keepdims=True)
    idx2 = jnp.min(jnp.where(biased2 == m2, iota, NUM_EXPERTS), axis=1, keepdims=True)
    s2 = jnp.max(jnp.where(iota == idx2, scores, neg_inf), axis=1, keepdims=True)

    ts_ref[...] = jnp.concatenate([s1, s2], axis=1)
    se_ref[...] = jnp.concatenate([idx1, idx2], axis=1)

    cnt = jnp.sum(
        (iota == idx1).astype(jnp.float32) + (iota == idx2).astype(jnp.float32),
        axis=0,
    )[None, :]

    @pl.when(i == 0)
    def _init():
        cnt_ref[...] = cnt

    @pl.when(i > 0)
    def _acc():
        cnt_ref[...] += cnt


def kernel(x, W, expert_bias):
    bias2d = expert_bias.reshape(1, NUM_EXPERTS)
    grid = (T // TT,)
    top_scores, selected, counts = pl.pallas_call(
        _router_body,
        grid=grid,
        in_specs=[
            pl.BlockSpec((TT, DIM), lambda i: (i, 0)),
            pl.BlockSpec((NUM_EXPERTS, DIM), lambda i: (0, 0)),
            pl.BlockSpec((1, NUM_EXPERTS), lambda i: (0, 0)),
        ],
        out_specs=[
            pl.BlockSpec((TT, TOP_K), lambda i: (i, 0)),
            pl.BlockSpec((TT, TOP_K), lambda i: (i, 0)),
            pl.BlockSpec((1, NUM_EXPERTS), lambda i: (0, 0)),
        ],
        out_shape=[
            jax.ShapeDtypeStruct((T, TOP_K), jnp.float32),
            jax.ShapeDtypeStruct((T, TOP_K), jnp.int32),
            jax.ShapeDtypeStruct((1, NUM_EXPERTS), jnp.float32),
        ],
    )(x, W, bias2d)
    return top_scores, selected, counts.reshape(NUM_EXPERTS)


# speedup vs baseline: 1.4467x; 1.4467x over previous
import jax
import jax.numpy as jnp
from jax import lax
from jax.experimental import pallas as pl

DIM = 2048
NUM_EXPERTS = 16
TOP_K = 2
T = 16384
TT = 1024


def _router_body(x_ref, w_ref, b_ref, ts_ref, se_ref, cnt_ref):
    i = pl.program_id(0)
    logits = lax.dot_general(
        w_ref[...], x_ref[...],
        dimension_numbers=(((1,), (1,)), ((), ())),
        preferred_element_type=jnp.float32,
    )
    scores = jax.nn.sigmoid(logits)
    biased = scores + b_ref[...]

    iota = lax.broadcasted_iota(jnp.int32, (NUM_EXPERTS, TT), 0)
    neg_inf = jnp.float32(-jnp.inf)

    m1 = jnp.max(biased, axis=0, keepdims=True)
    idx1 = jnp.min(jnp.where(biased == m1, iota, NUM_EXPERTS), axis=0, keepdims=True)
    sel1 = iota == idx1
    s1 = jnp.max(jnp.where(sel1, scores, neg_inf), axis=0, keepdims=True)

    biased2 = jnp.where(sel1, neg_inf, biased)
    m2 = jnp.max(biased2, axis=0, keepdims=True)
    idx2 = jnp.min(jnp.where(biased2 == m2, iota, NUM_EXPERTS), axis=0, keepdims=True)
    sel2 = iota == idx2
    s2 = jnp.max(jnp.where(sel2, scores, neg_inf), axis=0, keepdims=True)

    ts_ref[...] = jnp.concatenate([s1, s2], axis=0)
    se_ref[...] = jnp.concatenate([idx1, idx2], axis=0)

    onehot = sel1.astype(jnp.float32) + sel2.astype(jnp.float32)
    ones = jnp.ones((TT, 1), dtype=jnp.float32)
    cnt = lax.dot_general(
        onehot, ones,
        dimension_numbers=(((1,), (0,)), ((), ())),
        preferred_element_type=jnp.float32,
    )

    @pl.when(i == 0)
    def _init():
        cnt_ref[...] = cnt

    @pl.when(i > 0)
    def _acc():
        cnt_ref[...] += cnt


def kernel(x, W, expert_bias):
    bias2d = expert_bias.reshape(NUM_EXPERTS, 1)
    grid = (T // TT,)
    ts_t, se_t, counts = pl.pallas_call(
        _router_body,
        grid=grid,
        in_specs=[
            pl.BlockSpec((TT, DIM), lambda i: (i, 0)),
            pl.BlockSpec((NUM_EXPERTS, DIM), lambda i: (0, 0)),
            pl.BlockSpec((NUM_EXPERTS, 1), lambda i: (0, 0)),
        ],
        out_specs=[
            pl.BlockSpec((TOP_K, TT), lambda i: (0, i)),
            pl.BlockSpec((TOP_K, TT), lambda i: (0, i)),
            pl.BlockSpec((NUM_EXPERTS, 1), lambda i: (0, 0)),
        ],
        out_shape=[
            jax.ShapeDtypeStruct((TOP_K, T), jnp.float32),
            jax.ShapeDtypeStruct((TOP_K, T), jnp.int32),
            jax.ShapeDtypeStruct((NUM_EXPERTS, 1), jnp.float32),
        ],
    )(x, W, bias2d)
    return ts_t.T, se_t.T, counts.reshape(NUM_EXPERTS)
